# baseline (speedup 1.0000x reference)
import functools

import jax
import jax.numpy as jnp
from jax import lax
from jax.experimental import pallas as pl
from jax.experimental.pallas import tpu as pltpu
from jax.experimental.pallas import tpu_sc as plsc

_B = 1024
_D = 32
_R = 1000
_NIDX = 2 * _B

_NC = 2
_NS = 16
_NACT = 16
_BPT = _NIDX // _NACT
_CH = 8
_NCHUNK = _BPT // _CH

_sc_mesh = plsc.VectorSubcoreMesh(core_axis_name="c", subcore_axis_name="s")


@functools.partial(
    pl.kernel,
    mesh=_sc_mesh,
    out_type=jax.ShapeDtypeStruct((_D, _NIDX), jnp.float32),
    scratch_types=[
        pltpu.VMEM((_BPT,), jnp.int32),
        [pltpu.VMEM((_D, 128), jnp.float32) for _ in range(_CH)],
        [pltpu.VMEM((_D, 128), jnp.float32) for _ in range(_CH)],
        pltpu.VMEM((_D, _BPT), jnp.float32),
        pltpu.SemaphoreType.DMA,
        pltpu.SemaphoreType.DMA,
    ],
)
def _sc_gather(heads_hbm, tails_hbm, table_hbm, out_hbm,
               idx_v, blks0, blks1, cols_v, sem0, sem1):
    wid = lax.axis_index("s") * _NC + lax.axis_index("c")

    @pl.when(wid < _NACT)
    def _():
        half = _NACT // 2

        @pl.when(wid < half)
        def _():
            pltpu.sync_copy(heads_hbm.at[pl.ds(wid * _BPT, _BPT)], idx_v)

        @pl.when(wid >= half)
        def _():
            pltpu.sync_copy(tails_hbm.at[pl.ds((wid - half) * _BPT, _BPT)],
                            idx_v)

        def fire(c, blks, sem):
            v = idx_v[pl.ds(c, _CH)]
            for j in range(_CH):
                blk = lax.shift_right_logical(v[j], 7) * 128
                pltpu.async_copy(
                    table_hbm.at[:, pl.ds(pl.multiple_of(blk, 128), 128)],
                    blks[j], sem)

        def select(c, blks, sem):
            v = idx_v[pl.ds(c, _CH)]
            for j in range(_CH):
                pltpu.make_async_copy(
                    table_hbm.at[:, pl.ds(0, 128)], blks[j], sem).wait()
                q = v[j] & 127
                cols_v[pl.ds(0, 16), pl.ds(c + j, 1)] = (
                    blks[j][pl.ds(0, 16), pl.ds(q, 1)])
                cols_v[pl.ds(16, 16), pl.ds(c + j, 1)] = (
                    blks[j][pl.ds(16, 16), pl.ds(q, 1)])

        fire(0, blks0, sem0)
        fire(_CH, blks1, sem1)

        def pair(g, carry):
            c0 = g * (2 * _CH)
            select(c0, blks0, sem0)

            @pl.when(c0 + 2 * _CH < _BPT)
            def _():
                fire(c0 + 2 * _CH, blks0, sem0)

            select(c0 + _CH, blks1, sem1)

            @pl.when(c0 + 3 * _CH < _BPT)
            def _():
                fire(c0 + 3 * _CH, blks1, sem1)

            return carry

        lax.fori_loop(0, _NCHUNK // 2, pair, 0)
        pltpu.sync_copy(cols_v, out_hbm.at[:, pl.ds(wid * _BPT, _BPT)])


_BBL = 256


def _tc_distance_body(h_ref, t_ref, rel_ref, out_ref):
    d = h_ref[...] - t_ref[...]
    acc = jnp.abs(rel_ref[:, 0:1] + d[0:1, :])
    for k in range(1, _D):
        acc = acc + jnp.abs(rel_ref[:, k:k + 1] + d[k:k + 1, :])
    out_ref[...] = -acc


def _tc_distance(rows_t, rel):
    nblk = _B // _BBL
    return pl.pallas_call(
        _tc_distance_body,
        grid=(nblk,),
        in_specs=[
            pl.BlockSpec((_D, _BBL), lambda i: (0, i)),
            pl.BlockSpec((_D, _BBL), lambda i: (0, i + nblk)),
            pl.BlockSpec((_R, _D), lambda i: (0, 0)),
        ],
        out_specs=pl.BlockSpec((_R, _BBL), lambda i: (0, i)),
        out_shape=jax.ShapeDtypeStruct((_R, _B), jnp.float32),
    )(rows_t, rows_t, rel)


def kernel(heads, tails, entity_emb, relation_emb):
    table_t = entity_emb.T
    rows_t = _sc_gather(heads.astype(jnp.int32), tails.astype(jnp.int32),
                        table_t)
    out_t = _tc_distance(rows_t, relation_emb)
    return out_t.T


# speedup vs baseline: 1.0772x; 1.0772x over previous
import functools

import jax
import jax.numpy as jnp
from jax import lax
from jax.experimental import pallas as pl
from jax.experimental.pallas import tpu as pltpu
from jax.experimental.pallas import tpu_sc as plsc

_B = 1024
_D = 32
_R = 1000
_NIDX = 2 * _B

_NC = 2
_NS = 16
_NACT = 16
_BPT = _NIDX // _NACT
_CH = 8
_NCHUNK = _BPT // _CH

_sc_mesh = plsc.VectorSubcoreMesh(core_axis_name="c", subcore_axis_name="s")


@functools.partial(
    pl.kernel,
    mesh=_sc_mesh,
    out_type=jax.ShapeDtypeStruct((_D, _NIDX), jnp.float32),
    scratch_types=[
        pltpu.VMEM((_BPT,), jnp.int32),
        [pltpu.VMEM((_D, 128), jnp.float32) for _ in range(_CH)],
        [pltpu.VMEM((_D, 128), jnp.float32) for _ in range(_CH)],
        pltpu.VMEM((_D, _BPT), jnp.float32),
        pltpu.SemaphoreType.DMA,
        pltpu.SemaphoreType.DMA,
    ],
)
def _sc_gather(heads_hbm, tails_hbm, table_hbm, out_hbm,
               idx_v, blks0, blks1, cols_v, sem0, sem1):
    wid = lax.axis_index("s") * _NC + lax.axis_index("c")

    @pl.when(wid < _NACT)
    def _():
        half = _NACT // 2

        @pl.when(wid < half)
        def _():
            pltpu.sync_copy(heads_hbm.at[pl.ds(wid * _BPT, _BPT)], idx_v)

        @pl.when(wid >= half)
        def _():
            pltpu.sync_copy(tails_hbm.at[pl.ds((wid - half) * _BPT, _BPT)],
                            idx_v)

        def fire(c, blks, sem):
            v = idx_v[pl.ds(c, _CH)]
            for j in range(_CH):
                blk = lax.shift_right_logical(v[j], 7) * 128
                pltpu.async_copy(
                    table_hbm.at[:, pl.ds(pl.multiple_of(blk, 128), 128)],
                    blks[j], sem)

        def select(c, blks, sem):
            v = idx_v[pl.ds(c, _CH)]
            for j in range(_CH):
                pltpu.make_async_copy(
                    table_hbm.at[:, pl.ds(0, 128)], blks[j], sem).wait()
                q = v[j] & 127
                cols_v[pl.ds(0, 16), pl.ds(c + j, 1)] = (
                    blks[j][pl.ds(0, 16), pl.ds(q, 1)])
                cols_v[pl.ds(16, 16), pl.ds(c + j, 1)] = (
                    blks[j][pl.ds(16, 16), pl.ds(q, 1)])

        fire(0, blks0, sem0)
        fire(_CH, blks1, sem1)

        def pair(g, carry):
            c0 = g * (2 * _CH)
            select(c0, blks0, sem0)

            @pl.when(c0 + 2 * _CH < _BPT)
            def _():
                fire(c0 + 2 * _CH, blks0, sem0)

            select(c0 + _CH, blks1, sem1)

            @pl.when(c0 + 3 * _CH < _BPT)
            def _():
                fire(c0 + 3 * _CH, blks1, sem1)

            return carry

        lax.fori_loop(0, _NCHUNK // 2, pair, 0)
        pltpu.sync_copy(cols_v, out_hbm.at[:, pl.ds(wid * _BPT, _BPT)])


_BBL = 1024


def _tc_distance_body(h_ref, t_ref, rel_ref, out_ref):
    d = h_ref[...] - t_ref[...]
    acc = jnp.abs(rel_ref[:, 0:1] + d[0:1, :])
    for k in range(1, _D):
        acc = acc + jnp.abs(rel_ref[:, k:k + 1] + d[k:k + 1, :])
    out_ref[...] = -acc


def _tc_distance(rows_t, rel):
    nblk = _B // _BBL
    return pl.pallas_call(
        _tc_distance_body,
        grid=(nblk,),
        in_specs=[
            pl.BlockSpec((_D, _BBL), lambda i: (0, i)),
            pl.BlockSpec((_D, _BBL), lambda i: (0, i + nblk)),
            pl.BlockSpec((_R, _D), lambda i: (0, 0)),
        ],
        out_specs=pl.BlockSpec((_R, _BBL), lambda i: (0, i)),
        out_shape=jax.ShapeDtypeStruct((_R, _B), jnp.float32),
    )(rows_t, rows_t, rel)


def kernel(heads, tails, entity_emb, relation_emb):
    table_t = entity_emb.T
    rows_t = _sc_gather(heads.astype(jnp.int32), tails.astype(jnp.int32),
                        table_t)
    out_t = _tc_distance(rows_t, relation_emb)
    return out_t.T


# speedup vs baseline: 1.2749x; 1.1835x over previous
import functools

import jax
import jax.numpy as jnp
from jax import lax
from jax.experimental import pallas as pl
from jax.experimental.pallas import tpu as pltpu
from jax.experimental.pallas import tpu_sc as plsc

_B = 1024
_D = 32
_R = 1000
_NIDX = 2 * _B

_NC = 2
_NS = 16
_GPT = _NIDX // 16
_CH = 8

_sc_mesh = plsc.VectorSubcoreMesh(core_axis_name="c", subcore_axis_name="s")


@functools.partial(
    pl.kernel,
    mesh=_sc_mesh,
    out_type=jax.ShapeDtypeStruct((_D, _NIDX), jnp.float32),
    scratch_types=[
        pltpu.VMEM((_GPT,), jnp.int32),
        [pltpu.VMEM((_D // 2, 128), jnp.float32) for _ in range(_CH)],
        [pltpu.VMEM((_D // 2, 128), jnp.float32) for _ in range(_CH)],
        pltpu.VMEM((_D // 2, _GPT), jnp.float32),
        pltpu.SemaphoreType.DMA,
        pltpu.SemaphoreType.DMA,
    ],
)
def _sc_gather(heads_hbm, tails_hbm, table_hbm, out_hbm,
               idx_v, blks0, blks1, cols_v, sem0, sem1):
    wid = lax.axis_index("s") * _NC + lax.axis_index("c")
    group = wid % 16

    @pl.when(group < 8)
    def _():
        pltpu.sync_copy(heads_hbm.at[pl.ds(group * _GPT, _GPT)], idx_v)

    @pl.when(group >= 8)
    def _():
        pltpu.sync_copy(tails_hbm.at[pl.ds((group - 8) * _GPT, _GPT)], idx_v)

    def run(rowoff):
        def fire(c, blks, sem):
            v = idx_v[pl.ds(c, _CH)]
            for j in range(_CH):
                blk = lax.shift_right_logical(v[j], 7) * 128
                pltpu.async_copy(
                    table_hbm.at[pl.ds(rowoff, _D // 2),
                                 pl.ds(pl.multiple_of(blk, 128), 128)],
                    blks[j], sem)

        def select(c, blks, sem):
            v = idx_v[pl.ds(c, _CH)]
            for j in range(_CH):
                pltpu.make_async_copy(
                    table_hbm.at[pl.ds(rowoff, _D // 2), pl.ds(0, 128)],
                    blks[j], sem).wait()
                q = v[j] & 127
                cols_v[pl.ds(0, 16), pl.ds(c + j, 1)] = (
                    blks[j][pl.ds(0, 16), pl.ds(q, 1)])

        fire(0, blks0, sem0)
        fire(_CH, blks1, sem1)

        def pair(g, carry):
            c0 = g * (2 * _CH)
            select(c0, blks0, sem0)

            @pl.when(c0 + 2 * _CH < _GPT)
            def _():
                fire(c0 + 2 * _CH, blks0, sem0)

            select(c0 + _CH, blks1, sem1)

            @pl.when(c0 + 3 * _CH < _GPT)
            def _():
                fire(c0 + 3 * _CH, blks1, sem1)

            return carry

        lax.fori_loop(0, _GPT // (2 * _CH), pair, 0)
        pltpu.sync_copy(
            cols_v,
            out_hbm.at[pl.ds(rowoff, _D // 2), pl.ds(group * _GPT, _GPT)])

    @pl.when(wid < 16)
    def _():
        run(0)

    @pl.when(wid >= 16)
    def _():
        run(16)


_BBL = 1024


def _tc_distance_body(h_ref, t_ref, rel_ref, out_ref):
    d = h_ref[...] - t_ref[...]
    acc = jnp.abs(rel_ref[:, 0:1] + d[0:1, :])
    for k in range(1, _D):
        acc = acc + jnp.abs(rel_ref[:, k:k + 1] + d[k:k + 1, :])
    out_ref[...] = -acc


def _tc_distance(rows_t, rel):
    nblk = _B // _BBL
    return pl.pallas_call(
        _tc_distance_body,
        grid=(nblk,),
        in_specs=[
            pl.BlockSpec((_D, _BBL), lambda i: (0, i)),
            pl.BlockSpec((_D, _BBL), lambda i: (0, i + nblk)),
            pl.BlockSpec((_R, _D), lambda i: (0, 0)),
        ],
        out_specs=pl.BlockSpec((_R, _BBL), lambda i: (0, i)),
        out_shape=jax.ShapeDtypeStruct((_R, _B), jnp.float32),
    )(rows_t, rows_t, rel)


def kernel(heads, tails, entity_emb, relation_emb):
    table_t = entity_emb.T
    rows_t = _sc_gather(heads.astype(jnp.int32), tails.astype(jnp.int32),
                        table_t)
    out_t = _tc_distance(rows_t, relation_emb)
    return out_t.T


# speedup vs baseline: 1.2991x; 1.0190x over previous
import functools

import jax
import jax.numpy as jnp
from jax import lax
from jax.experimental import pallas as pl
from jax.experimental.pallas import tpu as pltpu
from jax.experimental.pallas import tpu_sc as plsc

_B = 1024
_D = 32
_R = 1000
_NIDX = 2 * _B

_NC = 2
_NS = 16
_GPT = _NIDX // 16
_CH = 8

_sc_mesh = plsc.VectorSubcoreMesh(core_axis_name="c", subcore_axis_name="s")


@functools.partial(
    pl.kernel,
    mesh=_sc_mesh,
    out_type=jax.ShapeDtypeStruct((_D, _NIDX), jnp.float32),
    scratch_types=[
        pltpu.VMEM((_GPT,), jnp.int32),
        [pltpu.VMEM((_D // 2, 128), jnp.float32) for _ in range(_CH)],
        [pltpu.VMEM((_D // 2, 128), jnp.float32) for _ in range(_CH)],
        pltpu.VMEM((_D // 2, _GPT), jnp.float32),
        pltpu.SemaphoreType.DMA,
        pltpu.SemaphoreType.DMA,
    ],
)
def _sc_gather(heads_hbm, tails_hbm, table_hbm, out_hbm,
               idx_v, blks0, blks1, cols_v, sem0, sem1):
    wid = lax.axis_index("s") * _NC + lax.axis_index("c")
    group = wid % 16

    @pl.when(group < 8)
    def _():
        pltpu.sync_copy(heads_hbm.at[pl.ds(group * _GPT, _GPT)], idx_v)

    @pl.when(group >= 8)
    def _():
        pltpu.sync_copy(tails_hbm.at[pl.ds((group - 8) * _GPT, _GPT)], idx_v)

    def run(rowoff):
        def fire(c, blks, sem):
            v = idx_v[pl.ds(c, _CH)]
            for j in range(_CH):
                blk = lax.shift_right_logical(v[j], 7) * 128
                pltpu.async_copy(
                    table_hbm.at[pl.ds(rowoff, _D // 2),
                                 pl.ds(pl.multiple_of(blk, 128), 128)],
                    blks[j], sem)

        def select(c, blks, sem):
            v = idx_v[pl.ds(c, _CH)]
            for j in range(_CH):
                pltpu.make_async_copy(
                    table_hbm.at[pl.ds(rowoff, _D // 2), pl.ds(0, 128)],
                    blks[j], sem).wait()
                q = v[j] & 127
                cols_v[pl.ds(0, 16), pl.ds(c + j, 1)] = (
                    blks[j][pl.ds(0, 16), pl.ds(q, 1)])

        fire(0, blks0, sem0)
        fire(_CH, blks1, sem1)

        def pair(g, carry):
            c0 = g * (2 * _CH)
            select(c0, blks0, sem0)

            @pl.when(c0 + 2 * _CH < _GPT)
            def _():
                fire(c0 + 2 * _CH, blks0, sem0)

            select(c0 + _CH, blks1, sem1)

            @pl.when(c0 + 3 * _CH < _GPT)
            def _():
                fire(c0 + 3 * _CH, blks1, sem1)

            return carry

        lax.fori_loop(0, _GPT // (2 * _CH), pair, 0)
        pltpu.sync_copy(
            cols_v,
            out_hbm.at[pl.ds(rowoff, _D // 2), pl.ds(group * _GPT, _GPT)])

    @pl.when(wid < 16)
    def _():
        run(0)

    @pl.when(wid >= 16)
    def _():
        run(16)


_BBL = 512


def _tc_distance_body(h_ref, t_ref, rel_ref, out_ref):
    d = h_ref[...] - t_ref[...]
    acc = jnp.abs(rel_ref[:, 0:1] + d[0:1, :])
    for k in range(1, _D):
        acc = acc + jnp.abs(rel_ref[:, k:k + 1] + d[k:k + 1, :])
    out_ref[...] = -acc


def _tc_distance(rows_t, rel):
    nblk = _B // _BBL
    return pl.pallas_call(
        _tc_distance_body,
        grid=(nblk,),
        in_specs=[
            pl.BlockSpec((_D, _BBL), lambda i: (0, i)),
            pl.BlockSpec((_D, _BBL), lambda i: (0, i + nblk)),
            pl.BlockSpec((_R, _D), lambda i: (0, 0)),
        ],
        out_specs=pl.BlockSpec((_R, _BBL), lambda i: (0, i)),
        out_shape=jax.ShapeDtypeStruct((_R, _B), jnp.float32),
    )(rows_t, rows_t, rel)


def kernel(heads, tails, entity_emb, relation_emb):
    table_t = entity_emb.T
    rows_t = _sc_gather(heads.astype(jnp.int32), tails.astype(jnp.int32),
                        table_t)
    out_t = _tc_distance(rows_t, relation_emb)
    return out_t.T


# speedup vs baseline: 1.4504x; 1.1164x over previous
import functools

import jax
import jax.numpy as jnp
from jax import lax
from jax.experimental import pallas as pl
from jax.experimental.pallas import tpu as pltpu
from jax.experimental.pallas import tpu_sc as plsc

_B = 1024
_D = 32
_R = 1000
_NIDX = 2 * _B

_NC = 2
_NS = 16
_GPT = _NIDX // 16
_CH = 8

_sc_mesh = plsc.VectorSubcoreMesh(core_axis_name="c", subcore_axis_name="s")


@functools.partial(
    pl.kernel,
    mesh=_sc_mesh,
    out_type=jax.ShapeDtypeStruct((_D, _NIDX), jnp.float32),
    scratch_types=[
        pltpu.VMEM((_GPT,), jnp.int32),
        [pltpu.VMEM((_D // 2, 128), jnp.float32) for _ in range(_CH)],
        [pltpu.VMEM((_D // 2, 128), jnp.float32) for _ in range(_CH)],
        pltpu.VMEM((_D // 2, _GPT), jnp.float32),
        pltpu.SemaphoreType.DMA,
        pltpu.SemaphoreType.DMA,
    ],
)
def _sc_gather(heads_hbm, tails_hbm, table_hbm, out_hbm,
               idx_v, blks0, blks1, cols_v, sem0, sem1):
    wid = lax.axis_index("s") * _NC + lax.axis_index("c")
    group = wid % 16

    @pl.when(group < 8)
    def _():
        pltpu.sync_copy(heads_hbm.at[pl.ds(group * _GPT, _GPT)], idx_v)

    @pl.when(group >= 8)
    def _():
        pltpu.sync_copy(tails_hbm.at[pl.ds((group - 8) * _GPT, _GPT)], idx_v)

    def run(rowoff):
        def fire(c, blks, sem):
            v = idx_v[pl.ds(c, _CH)]
            for j in range(_CH):
                blk = lax.shift_right_logical(v[j], 7) * 128
                pltpu.async_copy(
                    table_hbm.at[pl.ds(rowoff, _D // 2),
                                 pl.ds(pl.multiple_of(blk, 128), 128)],
                    blks[j], sem)

        def select(c, blks, sem):
            v = idx_v[pl.ds(c, _CH)]
            for j in range(_CH):
                pltpu.make_async_copy(
                    table_hbm.at[pl.ds(rowoff, _D // 2), pl.ds(0, 128)],
                    blks[j], sem).wait()
                q = v[j] & 127
                cols_v[pl.ds(0, 16), pl.ds(c + j, 1)] = (
                    blks[j][pl.ds(0, 16), pl.ds(q, 1)])

        fire(0, blks0, sem0)
        fire(_CH, blks1, sem1)

        def pair(g, carry):
            c0 = g * (2 * _CH)
            select(c0, blks0, sem0)

            @pl.when(c0 + 2 * _CH < _GPT)
            def _():
                fire(c0 + 2 * _CH, blks0, sem0)

            select(c0 + _CH, blks1, sem1)

            @pl.when(c0 + 3 * _CH < _GPT)
            def _():
                fire(c0 + 3 * _CH, blks1, sem1)

            return carry

        lax.fori_loop(0, _GPT // (2 * _CH), pair, 0)
        pltpu.sync_copy(
            cols_v,
            out_hbm.at[pl.ds(rowoff, _D // 2), pl.ds(group * _GPT, _GPT)])

    @pl.when(wid < 16)
    def _():
        run(0)

    @pl.when(wid >= 16)
    def _():
        run(16)


_BBL = 512


def _tc_distance_body(h_ref, t_ref, rel_ref, out_ref):
    d = (h_ref[...] - t_ref[...]).astype(jnp.bfloat16)
    rel = rel_ref[...].astype(jnp.bfloat16)
    acc = jnp.abs(rel[:, 0:1] + d[0:1, :])
    for k in range(1, _D):
        acc = acc + jnp.abs(rel[:, k:k + 1] + d[k:k + 1, :])
    out_ref[...] = -acc.astype(jnp.float32)


def _tc_distance(rows_t, rel):
    nblk = _B // _BBL
    return pl.pallas_call(
        _tc_distance_body,
        grid=(nblk,),
        in_specs=[
            pl.BlockSpec((_D, _BBL), lambda i: (0, i)),
            pl.BlockSpec((_D, _BBL), lambda i: (0, i + nblk)),
            pl.BlockSpec((_R, _D), lambda i: (0, 0)),
        ],
        out_specs=pl.BlockSpec((_R, _BBL), lambda i: (0, i)),
        out_shape=jax.ShapeDtypeStruct((_R, _B), jnp.float32),
    )(rows_t, rows_t, rel)


def kernel(heads, tails, entity_emb, relation_emb):
    table_t = entity_emb.T
    rows_t = _sc_gather(heads.astype(jnp.int32), tails.astype(jnp.int32),
                        table_t)
    out_t = _tc_distance(rows_t, relation_emb)
    return out_t.T
